# baseline (speedup 1.0000x reference)
import functools

import jax
import jax.numpy as jnp
from jax import lax
from jax.experimental import pallas as pl
from jax.experimental.pallas import tpu as pltpu
from jax.experimental.pallas import tpu_sc as plsc

_B = 128
_L = 16
_PAD = 160
_TOKENS = _B * (_B - 1) // 2
_D = 1024
_NC = 2
_NS = 16
_NW = _NC * _NS
_RPW = 256
_RLAST = _TOKENS - (_NW - 1) * _RPW

_mesh = plsc.VectorSubcoreMesh(core_axis_name="c", subcore_axis_name="s")


@functools.partial(
    pl.kernel,
    mesh=_mesh,
    out_type=(
        jax.ShapeDtypeStruct((_TOKENS, _D), jnp.float32),
        jax.ShapeDtypeStruct((_PAD,), jnp.int32),
    ),
    scratch_types=[
        pltpu.VMEM((_B,), jnp.int32),
        pltpu.VMEM((_PAD,), jnp.int32),
        pltpu.SemaphoreType.DMA,
    ],
    compiler_params=pltpu.CompilerParams(needs_layout_passes=False),
)
def _ragged_sc(values_hbm, rl_hbm, vout_hbm, splits_hbm, rl_v, out_v, sem):
    c = lax.axis_index("c")
    s = lax.axis_index("s")
    wid = s * _NC + c
    base = wid * _RPW
    cp_main = pltpu.make_async_copy(
        values_hbm.at[pl.ds(base, _RPW)], vout_hbm.at[pl.ds(base, _RPW)], sem
    )
    cp_last = pltpu.make_async_copy(
        values_hbm.at[pl.ds(base, _RLAST)], vout_hbm.at[pl.ds(base, _RLAST)], sem
    )

    @pl.when(wid < _NW - 1)
    def _():
        cp_main.start()

    @pl.when(wid == _NW - 1)
    def _():
        cp_last.start()

    @pl.when((c == 0) & (s == 0))
    def _():
        pltpu.sync_copy(rl_hbm, rl_v)
        carry = jnp.int32(0)
        for j in range(_B // _L):
            x = rl_v[pl.ds(j * _L, _L)]
            inc = plsc.cumsum(x)
            out_v[pl.ds(j * _L, _L)] = (inc - x) + carry
            carry = carry + jnp.sum(x)
        total = jnp.zeros((_L,), jnp.int32) + carry
        for j in range(_B // _L, _PAD // _L):
            out_v[pl.ds(j * _L, _L)] = total
        pltpu.sync_copy(out_v, splits_hbm)

    @pl.when(wid < _NW - 1)
    def _():
        cp_main.wait()

    @pl.when(wid == _NW - 1)
    def _():
        cp_last.wait()


def kernel(values, row_lengths):
    values_out, splits_padded = _ragged_sc(values, row_lengths)
    row_splits = lax.slice(splits_padded, (0,), (_B + 1,))
    return values_out, row_splits


# speedup vs baseline: 26.7800x; 26.7800x over previous
import functools

import jax
import jax.numpy as jnp
from jax import lax
from jax.experimental import pallas as pl
from jax.experimental.pallas import tpu as pltpu
from jax.experimental.pallas import tpu_sc as plsc

_B = 128
_L = 16
_PAD = 160
_TOKENS = _B * (_B - 1) // 2
_D = 1024
_BLK = 1016

_mesh = plsc.VectorSubcoreMesh(core_axis_name="c", subcore_axis_name="s")


@functools.partial(
    pl.kernel,
    mesh=_mesh,
    out_type=jax.ShapeDtypeStruct((_PAD,), jnp.int32),
    scratch_types=[
        pltpu.VMEM((_B,), jnp.int32),
        pltpu.VMEM((_PAD,), jnp.int32),
    ],
    compiler_params=pltpu.CompilerParams(needs_layout_passes=False),
)
def _row_splits_sc(rl_hbm, out_hbm, rl_v, out_v):
    @pl.when((lax.axis_index("c") == 0) & (lax.axis_index("s") == 0))
    def _():
        pltpu.sync_copy(rl_hbm, rl_v)
        carry = jnp.int32(0)
        for j in range(_B // _L):
            x = rl_v[pl.ds(j * _L, _L)]
            inc = plsc.cumsum(x)
            out_v[pl.ds(j * _L, _L)] = (inc - x) + carry
            carry = carry + jnp.sum(x)
        total = jnp.zeros((_L,), jnp.int32) + carry
        for j in range(_B // _L, _PAD // _L):
            out_v[pl.ds(j * _L, _L)] = total
        pltpu.sync_copy(out_v, out_hbm)


def _copy_body(src_ref, dst_ref):
    dst_ref[...] = src_ref[...]


_values_copy_tc = pl.pallas_call(
    _copy_body,
    grid=(_TOKENS // _BLK,),
    in_specs=[pl.BlockSpec((_BLK, _D), lambda i: (i, 0))],
    out_specs=pl.BlockSpec((_BLK, _D), lambda i: (i, 0)),
    out_shape=jax.ShapeDtypeStruct((_TOKENS, _D), jnp.float32),
)


def kernel(values, row_lengths):
    splits_padded = _row_splits_sc(row_lengths)
    row_splits = lax.slice(splits_padded, (0,), (_B + 1,))
    values_out = _values_copy_tc(values)
    return values_out, row_splits


# speedup vs baseline: 27.9177x; 1.0425x over previous
import functools

import jax
import jax.numpy as jnp
from jax import lax
from jax.experimental import pallas as pl
from jax.experimental.pallas import tpu as pltpu
from jax.experimental.pallas import tpu_sc as plsc

_B = 128
_L = 16
_PAD = 160
_TOKENS = _B * (_B - 1) // 2
_D = 1024
_BLK = 1016

_mesh = plsc.VectorSubcoreMesh(core_axis_name="c", subcore_axis_name="s", num_cores=1)


@functools.partial(
    pl.kernel,
    mesh=_mesh,
    out_type=jax.ShapeDtypeStruct((_PAD,), jnp.int32),
    scratch_types=[
        pltpu.VMEM((_B,), jnp.int32),
        pltpu.VMEM((_PAD,), jnp.int32),
    ],
    compiler_params=pltpu.CompilerParams(needs_layout_passes=False),
)
def _row_splits_sc(rl_hbm, out_hbm, rl_v, out_v):
    @pl.when((lax.axis_index("c") == 0) & (lax.axis_index("s") == 0))
    def _():
        pltpu.sync_copy(rl_hbm, rl_v)
        carry = jnp.int32(0)
        for j in range(_B // _L):
            x = rl_v[pl.ds(j * _L, _L)]
            inc = plsc.cumsum(x)
            out_v[pl.ds(j * _L, _L)] = (inc - x) + carry
            carry = carry + jnp.sum(x)
        total = jnp.zeros((_L,), jnp.int32) + carry
        for j in range(_B // _L, _PAD // _L):
            out_v[pl.ds(j * _L, _L)] = total
        pltpu.sync_copy(out_v, out_hbm)


def _copy_body(src_ref, dst_ref):
    dst_ref[...] = src_ref[...]


_values_copy_tc = pl.pallas_call(
    _copy_body,
    grid=(_TOKENS // _BLK,),
    in_specs=[pl.BlockSpec((_BLK, _D), lambda i: (i, 0))],
    out_specs=pl.BlockSpec((_BLK, _D), lambda i: (i, 0)),
    out_shape=jax.ShapeDtypeStruct((_TOKENS, _D), jnp.float32),
)


def kernel(values, row_lengths):
    splits_padded = _row_splits_sc(row_lengths)
    row_splits = lax.slice(splits_padded, (0,), (_B + 1,))
    values_out = _values_copy_tc(values)
    return values_out, row_splits


# speedup vs baseline: 42.3173x; 1.5158x over previous
import jax
import jax.numpy as jnp
from jax import lax
from jax.experimental import pallas as pl

_B = 128
_SPAD = 256
_TOKENS = _B * (_B - 1) // 2
_D = 1024
_BLK = 1016


def _fused_body(values_ref, rl_ref, vout_ref, splits_ref):
    vout_ref[...] = values_ref[...]

    @pl.when(pl.program_id(0) == 0)
    def _():
        rl_col = rl_ref[...]
        j = lax.broadcasted_iota(jnp.int32, (_B, _SPAD), 0)
        i = lax.broadcasted_iota(jnp.int32, (_B, _SPAD), 1)
        contrib = jnp.where(j < i, rl_col, 0)
        splits_ref[...] = jnp.sum(contrib, axis=0, keepdims=True)


_fused_tc = pl.pallas_call(
    _fused_body,
    grid=(_TOKENS // _BLK,),
    in_specs=[
        pl.BlockSpec((_BLK, _D), lambda i: (i, 0)),
        pl.BlockSpec((_B, 1), lambda i: (0, 0)),
    ],
    out_specs=[
        pl.BlockSpec((_BLK, _D), lambda i: (i, 0)),
        pl.BlockSpec((1, _SPAD), lambda i: (0, 0)),
    ],
    out_shape=[
        jax.ShapeDtypeStruct((_TOKENS, _D), jnp.float32),
        jax.ShapeDtypeStruct((1, _SPAD), jnp.int32),
    ],
)


def kernel(values, row_lengths):
    values_out, splits_pad = _fused_tc(values, row_lengths.reshape(_B, 1))
    row_splits = splits_pad.reshape(_SPAD)[: _B + 1]
    return values_out, row_splits
